# baseline (speedup 1.0000x reference)
import functools

import jax
import jax.numpy as jnp
from jax import lax
from jax.experimental import pallas as pl
from jax.experimental.pallas import tpu as pltpu
from jax.experimental.pallas import tpu_sc as plsc

N_NODES = 10000
N_EDGES = 160000
N_FEAT = 256
HIDDEN = 512
N_CLASSES = 128

NC = 2
NS = 16
CHUNK = 64
NB = 4
N_CHUNKS = N_EDGES // CHUNK
TSLAB = 160
N_CHUNKS_PAD = NS * TSLAB
SLABH = TSLAB // 4
STRIPE = 624
TAIL = N_NODES - NS * STRIPE
TAIL_OFF = NS * STRIPE

_mesh = plsc.VectorSubcoreMesh(core_axis_name="c", subcore_axis_name="s")


@functools.partial(
    pl.kernel,
    out_type=(
        jax.ShapeDtypeStruct((N_NODES, 128), jnp.float32),
        jax.ShapeDtypeStruct((N_NODES, 128), jnp.float32),
    ),
    mesh=_mesh,
    scratch_types=[
        pltpu.VMEM((SLABH, CHUNK), jnp.int32),
        pltpu.VMEM((SLABH, CHUNK), jnp.int32),
        pltpu.VMEM((CHUNK, 128), jnp.float32),
        pltpu.VMEM((CHUNK, 128), jnp.float32),
        pltpu.VMEM((CHUNK, 128), jnp.float32),
        pltpu.VMEM((CHUNK, 128), jnp.float32),
        pltpu.VMEM_SHARED((N_NODES, 128), jnp.float32),
        [pltpu.SemaphoreType.DMA] * NB,
    ],
)
def _prop_kernel(ya_hbm, yb_hbm, src2_hbm, dst2_hbm, za_hbm, zb_hbm,
                 sslab_v, dslab_v, rows0_v, rows1_v, rows2_v, rows3_v,
                 acc, gsems):
    c = lax.axis_index("c")
    s = lax.axis_index("s")
    r0 = s * STRIPE
    rows = [rows0_v, rows1_v, rows2_v, rows3_v]

    def run(y_hbm, out_hbm):
        pltpu.sync_copy(y_hbm.at[pl.ds(r0, STRIPE)],
                        acc.at[pl.ds(r0, STRIPE)])

        @pl.when(s == NS - 1)
        def _():
            pltpu.sync_copy(y_hbm.at[pl.ds(TAIL_OFF, TAIL)],
                            acc.at[pl.ds(TAIL_OFF, TAIL)])

        plsc.subcore_barrier()

        def gissue(k, b):
            pltpu.async_copy(y_hbm.at[sslab_v.at[k]], rows[b], gsems[b])

        def gwait(k, b):
            pltpu.make_async_copy(
                y_hbm.at[sslab_v.at[k]], rows[b], gsems[b]).wait()

        def scat(k, b):
            pltpu.sync_copy(rows[b], acc.at[dslab_v.at[k]], add=True)

        for h in range(TSLAB // SLABH):
            off = s * TSLAB + h * SLABH
            pltpu.sync_copy(src2_hbm.at[pl.ds(off, SLABH)], sslab_v)
            pltpu.sync_copy(dst2_hbm.at[pl.ds(off, SLABH)], dslab_v)

            trip = jnp.minimum(SLABH, jnp.maximum(0, N_CHUNKS - off))

            @pl.when(trip > 0)
            def _piece():
                for b in range(NB):
                    gissue(b, b)

                def stage(k, b):
                    gwait(k, b)
                    scat(k, b)

                    @pl.when(k + NB < trip)
                    def _():
                        gissue(k + NB, b)

                def body(jj, _):
                    for i in range(NB):
                        stage(jj * NB + i, i)
                    return 0

                lax.fori_loop(0, trip // NB, body, 0)

        plsc.subcore_barrier()
        pltpu.sync_copy(acc.at[pl.ds(r0, STRIPE)],
                        out_hbm.at[pl.ds(r0, STRIPE)])

        @pl.when(s == NS - 1)
        def _():
            pltpu.sync_copy(acc.at[pl.ds(TAIL_OFF, TAIL)],
                            out_hbm.at[pl.ds(TAIL_OFF, TAIL)])

    @pl.when(c == 0)
    def _():
        run(ya_hbm, za_hbm)

    @pl.when(c == 1)
    def _():
        run(yb_hbm, zb_hbm)

BLK = 1000
GRID = N_NODES // BLK


def _scale_body(dega, x, dinv16, y0a, y0b):
    dv16 = lax.rsqrt(dega[...][:, 0:16])
    dinv16[...] = dv16
    y = x[...] * dv16[:, 0:1]
    y0a[...] = y[:, :128]
    y0b[...] = y[:, 128:]


def _mm_body(za, zb, dinv16, W1a, W1b, b1, Wmu, Wlv, y1a, y1b):
    dv = dinv16[...][:, 0:1]
    xpa = za[...] * dv
    xpb = zb[...] * dv
    h = jnp.dot(xpa, W1a[...], preferred_element_type=jnp.float32)
    h = h + jnp.dot(xpb, W1b[...], preferred_element_type=jnp.float32)
    h = jax.nn.relu(h + b1[...])
    y1a[...] = jnp.dot(h, Wmu[...], preferred_element_type=jnp.float32) * dv
    y1b[...] = jnp.dot(h, Wlv[...], preferred_element_type=jnp.float32) * dv


def _out_body(z1a, z1b, dinv16, bmu, blv, mu, lv):
    dv = dinv16[...][:, 0:1]
    mu[...] = z1a[...] * dv + bmu[...]
    lv[...] = z1b[...] * dv + blv[...]


def _row_spec(cols):
    return pl.BlockSpec((BLK, cols), lambda i: (i, 0))


def _full_spec(r, cols):
    return pl.BlockSpec((r, cols), lambda i: (0, 0))


_scale_call = pl.pallas_call(
    _scale_body,
    grid=(GRID,),
    in_specs=[_row_spec(128), _row_spec(N_FEAT)],
    out_specs=[_row_spec(16), _row_spec(128), _row_spec(128)],
    out_shape=[
        jax.ShapeDtypeStruct((N_NODES, 16), jnp.float32),
        jax.ShapeDtypeStruct((N_NODES, 128), jnp.float32),
        jax.ShapeDtypeStruct((N_NODES, 128), jnp.float32),
    ],
)

_mm_call = pl.pallas_call(
    _mm_body,
    grid=(GRID,),
    in_specs=[
        _row_spec(128), _row_spec(128), _row_spec(16),
        _full_spec(128, HIDDEN), _full_spec(128, HIDDEN), _full_spec(1, HIDDEN),
        _full_spec(HIDDEN, 128), _full_spec(HIDDEN, 128),
    ],
    out_specs=[_row_spec(128), _row_spec(128)],
    out_shape=[
        jax.ShapeDtypeStruct((N_NODES, 128), jnp.float32),
        jax.ShapeDtypeStruct((N_NODES, 128), jnp.float32),
    ],
)

_out_call = pl.pallas_call(
    _out_body,
    grid=(GRID,),
    in_specs=[
        _row_spec(128), _row_spec(128), _row_spec(16),
        _full_spec(1, 128), _full_spec(1, 128),
    ],
    out_specs=[_row_spec(128), _row_spec(128)],
    out_shape=[
        jax.ShapeDtypeStruct((N_NODES, 128), jnp.float32),
        jax.ShapeDtypeStruct((N_NODES, 128), jnp.float32),
    ],
)


def kernel(x, W1, b1, Wmu, bmu, Wlv, blv, edge_index):
    pad = N_CHUNKS_PAD - N_CHUNKS
    src2 = jnp.pad(edge_index[0].reshape(N_CHUNKS, CHUNK), ((0, pad), (0, 0)))
    dst2 = jnp.pad(edge_index[1].reshape(N_CHUNKS, CHUNK), ((0, pad), (0, 0)))
    ones128 = jnp.ones((N_NODES, 128), jnp.float32)

    dega, _ = _prop_kernel(ones128, ones128, src2, dst2)
    dinv16, y0a, y0b = _scale_call(dega, x)
    z0a, z0b = _prop_kernel(y0a, y0b, src2, dst2)
    y1a, y1b = _mm_call(z0a, z0b, dinv16,
                        W1[:128, :], W1[128:, :], b1.reshape(1, HIDDEN),
                        Wmu, Wlv)
    z1a, z1b = _prop_kernel(y1a, y1b, src2, dst2)
    mu, lv = _out_call(z1a, z1b, dinv16,
                       bmu.reshape(1, N_CLASSES), blv.reshape(1, N_CLASSES))
    return (mu, lv)


# speedup vs baseline: 1.1727x; 1.1727x over previous
import functools

import jax
import jax.numpy as jnp
from jax import lax
from jax.experimental import pallas as pl
from jax.experimental.pallas import tpu as pltpu
from jax.experimental.pallas import tpu_sc as plsc

N_NODES = 10000
N_EDGES = 160000
N_FEAT = 256
HIDDEN = 512
N_CLASSES = 128

NC = 2
NS = 16
CHUNK = 64
NB = 4
N_CHUNKS = N_EDGES // CHUNK
TSLAB = 160
N_CHUNKS_PAD = NS * TSLAB
SLABH = TSLAB // 4
STRIPE = 624
TAIL = N_NODES - NS * STRIPE
TAIL_OFF = NS * STRIPE

_mesh = plsc.VectorSubcoreMesh(core_axis_name="c", subcore_axis_name="s")


DEG_CHUNK = 128
N_DEG_CHUNKS = N_EDGES // DEG_CHUNK
N_DEG_CHUNKS_PAD = 1280
DEG_SLAB = N_DEG_CHUNKS_PAD // (NC * NS)


@functools.partial(
    pl.kernel,
    out_type=(
        jax.ShapeDtypeStruct((N_NODES, 128), jnp.float32),
        jax.ShapeDtypeStruct((N_NODES, 128), jnp.float32),
    ),
    mesh=_mesh,
    scratch_types=[
        pltpu.VMEM((DEG_SLAB, DEG_CHUNK), jnp.int32),
        pltpu.VMEM((DEG_CHUNK, 128), jnp.float32),
        pltpu.VMEM_SHARED((N_NODES, 128), jnp.float32),
    ],
)
def _deg_kernel(dstd_hbm, ones_hbm, dega_hbm, degb_hbm, dslab_v, ones_v, acc):
    c = lax.axis_index("c")
    s = lax.axis_index("s")
    r0 = s * STRIPE

    pltpu.sync_copy(ones_hbm.at[pl.ds(0, DEG_CHUNK)], ones_v)
    pltpu.sync_copy(ones_hbm.at[pl.ds(r0, STRIPE)],
                    acc.at[pl.ds(r0, STRIPE)])

    @pl.when(s == NS - 1)
    def _():
        pltpu.sync_copy(ones_hbm.at[pl.ds(TAIL_OFF, TAIL)],
                        acc.at[pl.ds(TAIL_OFF, TAIL)])

    off = (s * NC + c) * DEG_SLAB
    pltpu.sync_copy(dstd_hbm.at[pl.ds(off, DEG_SLAB)], dslab_v)
    plsc.subcore_barrier()

    trip = jnp.minimum(DEG_SLAB, jnp.maximum(0, N_DEG_CHUNKS - off))

    def body(j, _):
        pltpu.sync_copy(ones_v, acc.at[dslab_v.at[j]], add=True)
        return 0

    lax.fori_loop(0, trip, body, 0)
    plsc.subcore_barrier()

    def writeout(out_hbm):
        pltpu.sync_copy(acc.at[pl.ds(r0, STRIPE)],
                        out_hbm.at[pl.ds(r0, STRIPE)])

        @pl.when(s == NS - 1)
        def _():
            pltpu.sync_copy(acc.at[pl.ds(TAIL_OFF, TAIL)],
                            out_hbm.at[pl.ds(TAIL_OFF, TAIL)])

    @pl.when(c == 0)
    def _():
        writeout(dega_hbm)

    @pl.when(c == 1)
    def _():
        writeout(degb_hbm)


@functools.partial(
    pl.kernel,
    out_type=(
        jax.ShapeDtypeStruct((N_NODES, 128), jnp.float32),
        jax.ShapeDtypeStruct((N_NODES, 128), jnp.float32),
    ),
    mesh=_mesh,
    scratch_types=[
        pltpu.VMEM((SLABH, CHUNK), jnp.int32),
        pltpu.VMEM((SLABH, CHUNK), jnp.int32),
        pltpu.VMEM((CHUNK, 128), jnp.float32),
        pltpu.VMEM((CHUNK, 128), jnp.float32),
        pltpu.VMEM((CHUNK, 128), jnp.float32),
        pltpu.VMEM((CHUNK, 128), jnp.float32),
        pltpu.VMEM_SHARED((N_NODES, 128), jnp.float32),
        [pltpu.SemaphoreType.DMA] * NB,
    ],
)
def _prop_kernel(ya_hbm, yb_hbm, src2_hbm, dst2_hbm, za_hbm, zb_hbm,
                 sslab_v, dslab_v, rows0_v, rows1_v, rows2_v, rows3_v,
                 acc, gsems):
    c = lax.axis_index("c")
    s = lax.axis_index("s")
    r0 = s * STRIPE
    rows = [rows0_v, rows1_v, rows2_v, rows3_v]

    def run(y_hbm, out_hbm):
        pltpu.sync_copy(y_hbm.at[pl.ds(r0, STRIPE)],
                        acc.at[pl.ds(r0, STRIPE)])

        @pl.when(s == NS - 1)
        def _():
            pltpu.sync_copy(y_hbm.at[pl.ds(TAIL_OFF, TAIL)],
                            acc.at[pl.ds(TAIL_OFF, TAIL)])

        plsc.subcore_barrier()

        def gissue(k, b):
            pltpu.async_copy(y_hbm.at[sslab_v.at[k]], rows[b], gsems[b])

        def gwait(k, b):
            pltpu.make_async_copy(
                y_hbm.at[sslab_v.at[k]], rows[b], gsems[b]).wait()

        def scat(k, b):
            pltpu.sync_copy(rows[b], acc.at[dslab_v.at[k]], add=True)

        for h in range(TSLAB // SLABH):
            off = s * TSLAB + h * SLABH
            pltpu.sync_copy(src2_hbm.at[pl.ds(off, SLABH)], sslab_v)
            pltpu.sync_copy(dst2_hbm.at[pl.ds(off, SLABH)], dslab_v)

            trip = jnp.minimum(SLABH, jnp.maximum(0, N_CHUNKS - off))

            @pl.when(trip > 0)
            def _piece():
                for b in range(NB):
                    gissue(b, b)

                def stage(k, b):
                    gwait(k, b)
                    scat(k, b)

                    @pl.when(k + NB < trip)
                    def _():
                        gissue(k + NB, b)

                def body(jj, _):
                    for i in range(NB):
                        stage(jj * NB + i, i)
                    return 0

                lax.fori_loop(0, trip // NB, body, 0)

        plsc.subcore_barrier()
        pltpu.sync_copy(acc.at[pl.ds(r0, STRIPE)],
                        out_hbm.at[pl.ds(r0, STRIPE)])

        @pl.when(s == NS - 1)
        def _():
            pltpu.sync_copy(acc.at[pl.ds(TAIL_OFF, TAIL)],
                            out_hbm.at[pl.ds(TAIL_OFF, TAIL)])

    @pl.when(c == 0)
    def _():
        run(ya_hbm, za_hbm)

    @pl.when(c == 1)
    def _():
        run(yb_hbm, zb_hbm)

BLK = 1000
GRID = N_NODES // BLK


def _scale_body(dega, degb, x, dinv16, y0a, y0b):
    dv16 = lax.rsqrt(dega[...][:, 0:16] + degb[...][:, 0:16] - 1.0)
    dinv16[...] = dv16
    y = x[...] * dv16[:, 0:1]
    y0a[...] = y[:, :128]
    y0b[...] = y[:, 128:]


def _mm_body(za, zb, dinv16, W1a, W1b, b1, Wmu, Wlv, y1a, y1b):
    dv = dinv16[...][:, 0:1]
    xpa = za[...] * dv
    xpb = zb[...] * dv
    h = jnp.dot(xpa, W1a[...], preferred_element_type=jnp.float32)
    h = h + jnp.dot(xpb, W1b[...], preferred_element_type=jnp.float32)
    h = jax.nn.relu(h + b1[...])
    y1a[...] = jnp.dot(h, Wmu[...], preferred_element_type=jnp.float32) * dv
    y1b[...] = jnp.dot(h, Wlv[...], preferred_element_type=jnp.float32) * dv


def _out_body(z1a, z1b, dinv16, bmu, blv, mu, lv):
    dv = dinv16[...][:, 0:1]
    mu[...] = z1a[...] * dv + bmu[...]
    lv[...] = z1b[...] * dv + blv[...]


def _row_spec(cols):
    return pl.BlockSpec((BLK, cols), lambda i: (i, 0))


def _full_spec(r, cols):
    return pl.BlockSpec((r, cols), lambda i: (0, 0))


_scale_call = pl.pallas_call(
    _scale_body,
    grid=(GRID,),
    in_specs=[_row_spec(128), _row_spec(128), _row_spec(N_FEAT)],
    out_specs=[_row_spec(16), _row_spec(128), _row_spec(128)],
    out_shape=[
        jax.ShapeDtypeStruct((N_NODES, 16), jnp.float32),
        jax.ShapeDtypeStruct((N_NODES, 128), jnp.float32),
        jax.ShapeDtypeStruct((N_NODES, 128), jnp.float32),
    ],
)

_mm_call = pl.pallas_call(
    _mm_body,
    grid=(GRID,),
    in_specs=[
        _row_spec(128), _row_spec(128), _row_spec(16),
        _full_spec(128, HIDDEN), _full_spec(128, HIDDEN), _full_spec(1, HIDDEN),
        _full_spec(HIDDEN, 128), _full_spec(HIDDEN, 128),
    ],
    out_specs=[_row_spec(128), _row_spec(128)],
    out_shape=[
        jax.ShapeDtypeStruct((N_NODES, 128), jnp.float32),
        jax.ShapeDtypeStruct((N_NODES, 128), jnp.float32),
    ],
)

_out_call = pl.pallas_call(
    _out_body,
    grid=(GRID,),
    in_specs=[
        _row_spec(128), _row_spec(128), _row_spec(16),
        _full_spec(1, 128), _full_spec(1, 128),
    ],
    out_specs=[_row_spec(128), _row_spec(128)],
    out_shape=[
        jax.ShapeDtypeStruct((N_NODES, 128), jnp.float32),
        jax.ShapeDtypeStruct((N_NODES, 128), jnp.float32),
    ],
)


def kernel(x, W1, b1, Wmu, bmu, Wlv, blv, edge_index):
    pad = N_CHUNKS_PAD - N_CHUNKS
    src2 = jnp.pad(edge_index[0].reshape(N_CHUNKS, CHUNK), ((0, pad), (0, 0)))
    dst2 = jnp.pad(edge_index[1].reshape(N_CHUNKS, CHUNK), ((0, pad), (0, 0)))
    ones128 = jnp.ones((N_NODES, 128), jnp.float32)
    dstd = jnp.pad(edge_index[1].reshape(N_DEG_CHUNKS, DEG_CHUNK),
                   ((0, N_DEG_CHUNKS_PAD - N_DEG_CHUNKS), (0, 0)))

    dega, degb = _deg_kernel(dstd, ones128)
    dinv16, y0a, y0b = _scale_call(dega, degb, x)
    z0a, z0b = _prop_kernel(y0a, y0b, src2, dst2)
    y1a, y1b = _mm_call(z0a, z0b, dinv16,
                        W1[:128, :], W1[128:, :], b1.reshape(1, HIDDEN),
                        Wmu, Wlv)
    z1a, z1b = _prop_kernel(y1a, y1b, src2, dst2)
    mu, lv = _out_call(z1a, z1b, dinv16,
                       bmu.reshape(1, N_CLASSES), blv.reshape(1, N_CLASSES))
    return (mu, lv)
